# Initial kernel scaffold; baseline (speedup 1.0000x reference)
import jax
import jax.numpy as jnp
from jax.experimental import pallas as pl


def kernel(x, pos, batch, x_skip, pos_skip, batch_skip, W1, b1, g1, beta1, W2, b2, g2, beta2):
    raise NotImplementedError("write your pallas kernel here")


# speedup vs baseline: 10.0833x; 10.0833x over previous
import jax
import jax.numpy as jnp
from jax.experimental import pallas as pl

N_COARSE = 4096
N_FINE = 16384
NIN = 256
NSKIP = 128
NOUT = 256
DIM = NIN + NSKIP
TILE = 256
GRID = N_FINE // TILE
EPS = 1e-5


def _pass_a(ps_ref, posT_ref, x_ref, xs_ref, w1a_ref, w1b_ref, b1_ref,
            h1_ref, st_ref):
    i = pl.program_id(0)
    ps = ps_ref[...]
    posT = posT_ref[...]
    q2 = jnp.sum(ps * ps, axis=1, keepdims=True)
    c2 = jnp.sum(posT * posT, axis=0, keepdims=True)
    d2 = q2 + c2 - 2.0 * jnp.dot(ps, posT, preferred_element_type=jnp.float32)

    iota = jax.lax.broadcasted_iota(jnp.int32, d2.shape, 1)
    cur = d2
    acc = jnp.zeros_like(d2)
    wsum = jnp.zeros((TILE, 1), jnp.float32)
    for _ in range(3):
        m = jnp.min(cur, axis=1, keepdims=True)
        idxk = jnp.min(jnp.where(cur == m, iota, N_COARSE), axis=1,
                       keepdims=True)
        hit = iota == idxk
        w = 1.0 / jnp.clip(jnp.maximum(m, 0.0), 1e-16, None)
        acc = acc + jnp.where(hit, w, 0.0)
        wsum = wsum + w
        cur = jnp.where(hit, jnp.inf, cur)
    acc = acc / wsum

    y = jnp.dot(acc, x_ref[...], preferred_element_type=jnp.float32)
    h = (jnp.dot(y, w1a_ref[...], preferred_element_type=jnp.float32)
         + jnp.dot(xs_ref[...], w1b_ref[...],
                   preferred_element_type=jnp.float32)
         + b1_ref[...])
    h = jnp.maximum(h, 0.0)
    h1_ref[...] = h
    part = jnp.concatenate([jnp.sum(h, axis=0, keepdims=True),
                            jnp.sum(h * h, axis=0, keepdims=True)], axis=0)

    @pl.when(i == 0)
    def _():
        st_ref[...] = part

    @pl.when(i != 0)
    def _():
        st_ref[...] = st_ref[...] + part


def _pass_b(h1_ref, st_ref, g1_ref, be1_ref, w2_ref, b2_ref, h2_ref, st2_ref):
    i = pl.program_id(0)
    mu = st_ref[0:1, :] * (1.0 / N_FINE)
    var = st_ref[1:2, :] * (1.0 / N_FINE) - mu * mu
    a = g1_ref[...] / jnp.sqrt(var + EPS)
    c = be1_ref[...] - mu * a
    hb = h1_ref[...] * a + c
    h2 = jnp.maximum(
        jnp.dot(hb, w2_ref[...], preferred_element_type=jnp.float32)
        + b2_ref[...], 0.0)
    h2_ref[...] = h2
    part = jnp.concatenate([jnp.sum(h2, axis=0, keepdims=True),
                            jnp.sum(h2 * h2, axis=0, keepdims=True)], axis=0)

    @pl.when(i == 0)
    def _():
        st2_ref[...] = part

    @pl.when(i != 0)
    def _():
        st2_ref[...] = st2_ref[...] + part


def _pass_c(h2_ref, st2_ref, g2_ref, be2_ref, out_ref):
    mu = st2_ref[0:1, :] * (1.0 / N_FINE)
    var = st2_ref[1:2, :] * (1.0 / N_FINE) - mu * mu
    a = g2_ref[...] / jnp.sqrt(var + EPS)
    c = be2_ref[...] - mu * a
    out_ref[...] = h2_ref[...] * a + c


def kernel(x, pos, batch, x_skip, pos_skip, batch_skip,
           W1, b1, g1, beta1, W2, b2, g2, beta2):
    posT8 = jnp.zeros((8, N_COARSE), jnp.float32).at[:3, :].set(pos.T)
    ps8 = jnp.zeros((N_FINE, 8), jnp.float32).at[:, :3].set(pos_skip)
    w1a = W1[:NIN, :]
    w1b = W1[NIN:, :]

    full = lambda shape: pl.BlockSpec(shape, lambda i: (0, 0))
    rows = lambda w: pl.BlockSpec((TILE, w), lambda i: (i, 0))

    h1, st1 = pl.pallas_call(
        _pass_a,
        grid=(GRID,),
        in_specs=[rows(8), full((8, N_COARSE)), full((N_COARSE, NIN)),
                  rows(NSKIP), full((NIN, DIM)), full((NSKIP, DIM)),
                  full((1, DIM))],
        out_specs=[rows(DIM), full((2, DIM))],
        out_shape=[jax.ShapeDtypeStruct((N_FINE, DIM), jnp.float32),
                   jax.ShapeDtypeStruct((2, DIM), jnp.float32)],
    )(ps8, posT8, x, x_skip, w1a, w1b, b1.reshape(1, DIM))

    h2, st2 = pl.pallas_call(
        _pass_b,
        grid=(GRID,),
        in_specs=[rows(DIM), full((2, DIM)), full((1, DIM)), full((1, DIM)),
                  full((DIM, NOUT)), full((1, NOUT))],
        out_specs=[rows(NOUT), full((2, NOUT))],
        out_shape=[jax.ShapeDtypeStruct((N_FINE, NOUT), jnp.float32),
                   jax.ShapeDtypeStruct((2, NOUT), jnp.float32)],
    )(h1, st1, g1.reshape(1, DIM), beta1.reshape(1, DIM), W2,
      b2.reshape(1, NOUT))

    out = pl.pallas_call(
        _pass_c,
        grid=(GRID,),
        in_specs=[rows(NOUT), full((2, NOUT)), full((1, NOUT)),
                  full((1, NOUT))],
        out_specs=rows(NOUT),
        out_shape=jax.ShapeDtypeStruct((N_FINE, NOUT), jnp.float32),
    )(h2, st2, g2.reshape(1, NOUT), beta2.reshape(1, NOUT))

    return (out, pos_skip, batch_skip)
